# Initial kernel scaffold; baseline (speedup 1.0000x reference)
import functools

import jax
import jax.numpy as jnp
from jax.experimental import pallas as pl
from jax.experimental.pallas import tpu as pltpu

N = 50000
H = 64
L = 3
BN = 2000
NBLK = N // BN


def _encode_body(xt_ref, xc_ref, wt_ref, bt_ref, wc_ref, bc_ref, ht_ref, hc_ref):
    xt = xt_ref[...]
    xc = xc_ref[...]
    ht_ref[...] = jnp.dot(xt, wt_ref[...], preferred_element_type=jnp.float32) + bt_ref[...]
    hc_ref[...] = jnp.dot(xc, wc_ref[...], preferred_element_type=jnp.float32) + bc_ref[...]


def _encode(x_table, x_column, enc_t_W, enc_t_b, enc_c_W, enc_c_b):
    xt = x_table.reshape(N, 1).astype(jnp.float32)
    xc = x_column.reshape(N, 1).astype(jnp.float32)
    bt = enc_t_b.reshape(1, H)
    bc = enc_c_b.reshape(1, H)
    out_shape = [jax.ShapeDtypeStruct((N, H), jnp.float32)] * 2
    return pl.pallas_call(
        _encode_body,
        grid=(NBLK,),
        in_specs=[
            pl.BlockSpec((BN, 1), lambda i: (i, 0)),
            pl.BlockSpec((BN, 1), lambda i: (i, 0)),
            pl.BlockSpec((1, H), lambda i: (0, 0)),
            pl.BlockSpec((1, H), lambda i: (0, 0)),
            pl.BlockSpec((1, H), lambda i: (0, 0)),
            pl.BlockSpec((1, H), lambda i: (0, 0)),
        ],
        out_specs=[
            pl.BlockSpec((BN, H), lambda i: (i, 0)),
            pl.BlockSpec((BN, H), lambda i: (i, 0)),
        ],
        out_shape=out_shape,
    )(xt, xc, enc_t_W, bt, enc_c_W, bc)


def _combine_body(final, ht_ref, hc_ref, sc_ref, sf_ref, sr_ref, cnt_ref,
                  wl_ref, bl_ref, wr_ref, cwt_ref, cbt_ref, cwc_ref, cbc_ref,
                  ot_ref, oc_ref):
    ht = ht_ref[...]
    hc = hc_ref[...]
    cnt = cnt_ref[...]
    rc = 1.0 / jnp.maximum(cnt, 1.0)
    mc = sc_ref[...] * rc[0][:, None]
    mf = sf_ref[...] * rc[1][:, None]
    mr = sr_ref[...] * rc[2][:, None]
    dot = functools.partial(jnp.dot, preferred_element_type=jnp.float32)
    col_new = 0.5 * (dot(mc, wl_ref[0]) + dot(mf, wl_ref[1])
                     + dot(hc, wr_ref[0] + wr_ref[1])
                     + bl_ref[0, 0][None, :] + bl_ref[0, 1][None, :])
    tab_new = (dot(mr, wl_ref[2]) + dot(ht, wr_ref[2]) + bl_ref[0, 2][None, :])
    ht_new = jnp.maximum(ht + col_new, 0.0)
    hc_new = jnp.maximum(hc + tab_new, 0.0)
    if final:
        to = dot(ht_new, cwt_ref[...]) + cbt_ref[...]
        co = dot(hc_new, cwc_ref[...]) + cbc_ref[...]
        ot_ref[...] = 1.0 / (1.0 + jnp.exp(-to))
        oc_ref[...] = 1.0 / (1.0 + jnp.exp(-co))
    else:
        ot_ref[...] = ht_new
        oc_ref[...] = hc_new


def _combine(final, ht, hc, sum_c, sum_f, sum_r, cnt, Wl_l, bl_l, Wr_l,
             cls_t_W, cls_t_b, cls_c_W, cls_c_b):
    ow = 1 if final else H
    out_shape = [jax.ShapeDtypeStruct((N, ow), jnp.float32)] * 2
    bl2 = bl_l.reshape(1, 3, H)
    return pl.pallas_call(
        functools.partial(_combine_body, final),
        grid=(NBLK,),
        in_specs=[
            pl.BlockSpec((BN, H), lambda i: (i, 0)),
            pl.BlockSpec((BN, H), lambda i: (i, 0)),
            pl.BlockSpec((BN, H), lambda i: (i, 0)),
            pl.BlockSpec((BN, H), lambda i: (i, 0)),
            pl.BlockSpec((BN, H), lambda i: (i, 0)),
            pl.BlockSpec((3, BN), lambda i: (0, i)),
            pl.BlockSpec((3, H, H), lambda i: (0, 0, 0)),
            pl.BlockSpec((1, 3, H), lambda i: (0, 0, 0)),
            pl.BlockSpec((3, H, H), lambda i: (0, 0, 0)),
            pl.BlockSpec((H, 1), lambda i: (0, 0)),
            pl.BlockSpec((1, 1), lambda i: (0, 0)),
            pl.BlockSpec((H, 1), lambda i: (0, 0)),
            pl.BlockSpec((1, 1), lambda i: (0, 0)),
        ],
        out_specs=[
            pl.BlockSpec((BN, ow), lambda i: (i, 0)),
            pl.BlockSpec((BN, ow), lambda i: (i, 0)),
        ],
        out_shape=out_shape,
    )(ht, hc, sum_c, sum_f, sum_r, cnt, Wl_l, bl2, Wr_l,
      cls_t_W, cls_t_b.reshape(1, 1), cls_c_W, cls_c_b.reshape(1, 1))


def _seg_sums(h_t, h_c, ei_contains, ei_foreign_key, ei_rev_contains):
    def agg(x_src, ei):
        msgs = jnp.take(x_src, ei[0], axis=0)
        return jax.ops.segment_sum(msgs, ei[1], num_segments=N)
    return (agg(h_t, ei_contains), agg(h_c, ei_foreign_key),
            agg(h_c, ei_rev_contains))


def _counts(ei_contains, ei_foreign_key, ei_rev_contains):
    def cnt(ei):
        return jax.ops.segment_sum(jnp.ones(ei.shape[1], jnp.float32), ei[1],
                                   num_segments=N)
    return jnp.stack([cnt(ei_contains), cnt(ei_foreign_key),
                      cnt(ei_rev_contains)])


def kernel(x_table, x_column, ei_contains, ei_foreign_key, ei_rev_contains,
           enc_t_W, enc_t_b, enc_c_W, enc_c_b, Wl, bl, Wr,
           cls_t_W, cls_t_b, cls_c_W, cls_c_b):
    h_t, h_c = _encode(x_table, x_column, enc_t_W, enc_t_b, enc_c_W, enc_c_b)
    cnt = _counts(ei_contains, ei_foreign_key, ei_rev_contains)
    for l in range(L):
        sum_c, sum_f, sum_r = _seg_sums(h_t, h_c, ei_contains,
                                        ei_foreign_key, ei_rev_contains)
        h_t, h_c = _combine(l == L - 1, h_t, h_c, sum_c, sum_f, sum_r, cnt,
                            Wl[l], bl[l], Wr[l],
                            cls_t_W, cls_t_b.reshape(1,), cls_c_W, cls_c_b)
    return (h_t[:, 0], h_c[:, 0])


# speedup vs baseline: 1.0872x; 1.0872x over previous
import functools

import jax
import jax.numpy as jnp
from jax.experimental import pallas as pl
from jax.experimental.pallas import tpu as pltpu

N = 50000
H = 64
L = 3
BN = 2000
NBLK = N // BN


def _encode_body(xt_ref, xc_ref, wt_ref, bt_ref, wc_ref, bc_ref, ht_ref, hc_ref):
    xt = xt_ref[...]
    xc = xc_ref[...]
    ht_ref[...] = jnp.dot(xt, wt_ref[...], preferred_element_type=jnp.float32) + bt_ref[...]
    hc_ref[...] = jnp.dot(xc, wc_ref[...], preferred_element_type=jnp.float32) + bc_ref[...]


def _encode(x_table, x_column, enc_t_W, enc_t_b, enc_c_W, enc_c_b):
    xt = x_table.reshape(N, 1).astype(jnp.float32)
    xc = x_column.reshape(N, 1).astype(jnp.float32)
    bt = enc_t_b.reshape(1, H)
    bc = enc_c_b.reshape(1, H)
    out_shape = [jax.ShapeDtypeStruct((N, H), jnp.float32)] * 2
    return pl.pallas_call(
        _encode_body,
        grid=(NBLK,),
        in_specs=[
            pl.BlockSpec((BN, 1), lambda i: (i, 0)),
            pl.BlockSpec((BN, 1), lambda i: (i, 0)),
            pl.BlockSpec((1, H), lambda i: (0, 0)),
            pl.BlockSpec((1, H), lambda i: (0, 0)),
            pl.BlockSpec((1, H), lambda i: (0, 0)),
            pl.BlockSpec((1, H), lambda i: (0, 0)),
        ],
        out_specs=[
            pl.BlockSpec((BN, H), lambda i: (i, 0)),
            pl.BlockSpec((BN, H), lambda i: (i, 0)),
        ],
        out_shape=out_shape,
    )(xt, xc, enc_t_W, bt, enc_c_W, bc)


def _combine_body(final, ht_ref, hc_ref, sc_ref, sf_ref, sr_ref, cnt_ref,
                  wl_ref, bl_ref, wr_ref, cwt_ref, cbt_ref, cwc_ref, cbc_ref,
                  ot_ref, oc_ref):
    ht = ht_ref[...]
    hc = hc_ref[...]
    cnt = cnt_ref[...]
    rc = 1.0 / jnp.maximum(cnt, 1.0)
    mc = sc_ref[...] * rc[:, 0:1]
    mf = sf_ref[...] * rc[:, 1:2]
    mr = sr_ref[...] * rc[:, 2:3]
    dot = functools.partial(jnp.dot, preferred_element_type=jnp.float32)
    col_new = 0.5 * (dot(mc, wl_ref[0]) + dot(mf, wl_ref[1])
                     + dot(hc, wr_ref[0] + wr_ref[1])
                     + bl_ref[0, 0][None, :] + bl_ref[0, 1][None, :])
    tab_new = (dot(mr, wl_ref[2]) + dot(ht, wr_ref[2]) + bl_ref[0, 2][None, :])
    ht_new = jnp.maximum(ht + col_new, 0.0)
    hc_new = jnp.maximum(hc + tab_new, 0.0)
    if final:
        to = dot(ht_new, cwt_ref[...]) + cbt_ref[...]
        co = dot(hc_new, cwc_ref[...]) + cbc_ref[...]
        ot_ref[...] = 1.0 / (1.0 + jnp.exp(-to))
        oc_ref[...] = 1.0 / (1.0 + jnp.exp(-co))
    else:
        ot_ref[...] = ht_new
        oc_ref[...] = hc_new


def _combine(final, ht, hc, sum_c, sum_f, sum_r, cnt, Wl_l, bl_l, Wr_l,
             cls_t_W, cls_t_b, cls_c_W, cls_c_b):
    ow = 1 if final else H
    out_shape = [jax.ShapeDtypeStruct((N, ow), jnp.float32)] * 2
    bl2 = bl_l.reshape(1, 3, H)
    return pl.pallas_call(
        functools.partial(_combine_body, final),
        grid=(NBLK,),
        in_specs=[
            pl.BlockSpec((BN, H), lambda i: (i, 0)),
            pl.BlockSpec((BN, H), lambda i: (i, 0)),
            pl.BlockSpec((BN, H), lambda i: (i, 0)),
            pl.BlockSpec((BN, H), lambda i: (i, 0)),
            pl.BlockSpec((BN, H), lambda i: (i, 0)),
            pl.BlockSpec((BN, 3), lambda i: (i, 0)),
            pl.BlockSpec((3, H, H), lambda i: (0, 0, 0)),
            pl.BlockSpec((1, 3, H), lambda i: (0, 0, 0)),
            pl.BlockSpec((3, H, H), lambda i: (0, 0, 0)),
            pl.BlockSpec((H, 1), lambda i: (0, 0)),
            pl.BlockSpec((1, 1), lambda i: (0, 0)),
            pl.BlockSpec((H, 1), lambda i: (0, 0)),
            pl.BlockSpec((1, 1), lambda i: (0, 0)),
        ],
        out_specs=[
            pl.BlockSpec((BN, ow), lambda i: (i, 0)),
            pl.BlockSpec((BN, ow), lambda i: (i, 0)),
        ],
        out_shape=out_shape,
    )(ht, hc, sum_c, sum_f, sum_r, cnt, Wl_l, bl2, Wr_l,
      cls_t_W, cls_t_b.reshape(1, 1), cls_c_W, cls_c_b.reshape(1, 1))


def _seg_sums(h_t, h_c, ei_contains, ei_foreign_key, ei_rev_contains):
    def agg(x_src, ei):
        msgs = jnp.take(x_src, ei[0], axis=0)
        return jax.ops.segment_sum(msgs, ei[1], num_segments=N)
    return (agg(h_t, ei_contains), agg(h_c, ei_foreign_key),
            agg(h_c, ei_rev_contains))


def _counts(ei_contains, ei_foreign_key, ei_rev_contains):
    def cnt(ei):
        return jax.ops.segment_sum(jnp.ones(ei.shape[1], jnp.float32), ei[1],
                                   num_segments=N)
    return jnp.stack([cnt(ei_contains), cnt(ei_foreign_key),
                      cnt(ei_rev_contains)], axis=1)


def kernel(x_table, x_column, ei_contains, ei_foreign_key, ei_rev_contains,
           enc_t_W, enc_t_b, enc_c_W, enc_c_b, Wl, bl, Wr,
           cls_t_W, cls_t_b, cls_c_W, cls_c_b):
    h_t, h_c = _encode(x_table, x_column, enc_t_W, enc_t_b, enc_c_W, enc_c_b)
    cnt = _counts(ei_contains, ei_foreign_key, ei_rev_contains)
    for l in range(L):
        sum_c, sum_f, sum_r = _seg_sums(h_t, h_c, ei_contains,
                                        ei_foreign_key, ei_rev_contains)
        h_t, h_c = _combine(l == L - 1, h_t, h_c, sum_c, sum_f, sum_r, cnt,
                            Wl[l], bl[l], Wr[l],
                            cls_t_W, cls_t_b.reshape(1,), cls_c_W, cls_c_b)
    return (h_t[:, 0], h_c[:, 0])


# speedup vs baseline: 6.7498x; 6.2086x over previous
import functools

import jax
import jax.numpy as jnp
from jax import lax
from jax.experimental import pallas as pl
from jax.experimental.pallas import tpu as pltpu
from jax.experimental.pallas import tpu_sc as plsc

N = 50000
H = 64
HH = H // 2
L = 3
BN = 2000
NBLK = N // BN

NS = 16
NC = 2
IW = 128
CR = 8
CE = CR * IW
R_FK = 6256
R_C = 400
NP = 50176
TROWS = NP // NS
FZ = 112
NFZ = TROWS // FZ
CW = 3136
CW_LAST = N - 15 * CW


def _agg_body(with_counts, ht, hc, cs, cd, fs, fd, rs, rd, *rest):
    if with_counts:
        out_c, out_f, out_r, cnt_out = rest[:4]
        rest = rest[4:]
    else:
        out_c, out_f, out_r = rest[:3]
        rest = rest[3:]
    (sbuf, dbuf, r0, r1, r2, r3, stage, ones, cbuf,
     g0, g1, g2, g3, s0, s1, s2, s3, csem, acc, acc_cnt) = rest
    rows = [r0, r1, r2, r3]
    gsems = [g0, g1, g2, g3]
    ssems = [s0, s1, s2, s3]
    NB = 4

    c = lax.axis_index("c")
    s = lax.axis_index("s")

    def zero_stage():
        def zrow(r, carry):
            stage[r, pl.ds(0, 16)] = jnp.zeros((16,), jnp.float32)
            stage[r, pl.ds(16, 16)] = jnp.zeros((16,), jnp.float32)
            return carry
        lax.fori_loop(0, FZ, zrow, 0)

    if with_counts:
        for k in range(IW // 16):
            ones[0, pl.ds(16 * k, 16)] = jnp.ones((16,), jnp.float32)

    def do_rel(r_id, src_hbm, dst_hbm, src_tab, out_hbm, nrows, count_it,
               cnt_out=None):
        zero_stage()
        for k in range(TROWS // FZ):
            pltpu.sync_copy(stage, acc.at[pl.ds(s * TROWS + k * FZ, FZ)])
        if count_it:
            def czrow(i, carry):
                cbuf[pl.ds(16 * i, 16)] = jnp.zeros((16,), jnp.float32)
                return carry
            lax.fori_loop(0, CW // 16, czrow, 0)

            @pl.when(s < NS - 1)
            def _():
                pltpu.sync_copy(cbuf, acc_cnt.at[pl.ds(s * CW, CW)])

            @pl.when(s == NS - 1)
            def _():
                pltpu.sync_copy(cbuf.at[pl.ds(0, CW_LAST)],
                                acc_cnt.at[pl.ds((NS - 1) * CW, CW_LAST)])
                pltpu.sync_copy(cbuf.at[pl.ds(0, 8)],
                                acc_cnt.at[pl.ds(N, 8)])
        plsc.subcore_barrier()

        nch = nrows // CR
        iters = (nch + NS - 1) // NS

        def chunk(i, carry):
            j = s + i * NS

            @pl.when(j < nch)
            def _():
                base = j * CR
                pltpu.sync_copy(src_hbm.at[pl.ds(base, CR)], sbuf)
                pltpu.sync_copy(dst_hbm.at[pl.ds(base, CR)], dbuf)
                gd = {}
                sd = {}
                cd = {}
                for k in range(NB):
                    gd[k] = pltpu.async_copy(src_tab.at[c].at[sbuf.at[k]],
                                             rows[k], gsems[k])
                for k in range(CR):
                    b = k % NB
                    gd[k].wait()
                    sd[k] = pltpu.async_copy(rows[b], acc.at[dbuf.at[k]],
                                             ssems[b], add=True)
                    if count_it:
                        cd[k] = pltpu.async_copy(ones.at[0],
                                                 acc_cnt.at[dbuf.at[k]],
                                                 csem, add=True)
                    if k + NB < CR:
                        sd[k].wait()
                        gd[k + NB] = pltpu.async_copy(
                            src_tab.at[c].at[sbuf.at[k + NB]], rows[b],
                            gsems[b])
                for k in range(CR - NB, CR):
                    sd[k].wait()
                if count_it:
                    for k in range(CR):
                        cd[k].wait()
            return carry
        lax.fori_loop(0, iters, chunk, 0)
        plsc.subcore_barrier()

        for k in range(NFZ):
            row0 = s * TROWS + k * FZ
            if k < NFZ - 2:
                pltpu.sync_copy(acc.at[pl.ds(row0, FZ)], stage)
                pltpu.sync_copy(stage, out_hbm.at[c].at[pl.ds(row0, FZ)])
            elif k == NFZ - 2:
                @pl.when(s < NS - 1)
                def _():
                    pltpu.sync_copy(acc.at[pl.ds(row0, FZ)], stage)
                    pltpu.sync_copy(stage, out_hbm.at[c].at[pl.ds(row0, FZ)])

                @pl.when(s == NS - 1)
                def _():
                    pltpu.sync_copy(acc.at[pl.ds(row0, 48)],
                                    stage.at[pl.ds(0, 48)])
                    pltpu.sync_copy(stage.at[pl.ds(0, 48)],
                                    out_hbm.at[c].at[pl.ds(row0, 48)])
            else:
                @pl.when(s < NS - 1)
                def _():
                    pltpu.sync_copy(acc.at[pl.ds(row0, FZ)], stage)
                    pltpu.sync_copy(stage, out_hbm.at[c].at[pl.ds(row0, FZ)])
        if count_it:
            @pl.when((c == 0) & (s < NS - 1))
            def _():
                pltpu.sync_copy(acc_cnt.at[pl.ds(s * CW, CW)], cbuf)
                pltpu.sync_copy(cbuf, cnt_out.at[pl.ds(r_id * N + s * CW, CW)])

            @pl.when((c == 0) & (s == NS - 1))
            def _():
                pltpu.sync_copy(acc_cnt.at[pl.ds((NS - 1) * CW, CW_LAST)],
                                cbuf.at[pl.ds(0, CW_LAST)])
                pltpu.sync_copy(cbuf.at[pl.ds(0, CW_LAST)],
                                cnt_out.at[pl.ds(r_id * N + (NS - 1) * CW,
                                                 CW_LAST)])

    if with_counts:
        do_rel(0, cs, cd, ht, out_c, R_C, True, cnt_out)
        do_rel(1, fs, fd, hc, out_f, R_FK, True, cnt_out)
        do_rel(2, rs, rd, hc, out_r, R_C, True, cnt_out)
    else:
        do_rel(0, cs, cd, ht, out_c, R_C, False)
        do_rel(1, fs, fd, hc, out_f, R_FK, False)
        do_rel(2, rs, rd, hc, out_r, R_C, False)

    return None


def _sc_agg(with_counts, ht2, hc2, ec, ef, er):
    outs = [jax.ShapeDtypeStruct((NC, N, HH), jnp.float32)] * 3
    if with_counts:
        outs.append(jax.ShapeDtypeStruct((3 * N,), jnp.float32))
    scratch = (
        [pltpu.VMEM((CR, IW), jnp.int32)] * 2
        + [pltpu.VMEM((IW, HH), jnp.float32)] * 4
        + [pltpu.VMEM((FZ, HH), jnp.float32),
           pltpu.VMEM((1, IW), jnp.float32),
           pltpu.VMEM((CW,), jnp.float32)]
        + [pltpu.SemaphoreType.DMA] * 9
        + [pltpu.VMEM_SHARED((NP, HH), jnp.float32),
           pltpu.VMEM_SHARED((N + 8,), jnp.float32)]
    )
    fn = pl.kernel(
        functools.partial(_agg_body, with_counts),
        out_type=tuple(outs),
        mesh=plsc.VectorSubcoreMesh(core_axis_name="c", subcore_axis_name="s"),
        scratch_types=scratch,
        compiler_params=pltpu.CompilerParams(use_tc_tiling_on_sc=False),
    )
    return fn(ht2, hc2, ec[0], ec[1], ef[0], ef[1], er[0], er[1])


def _prep_edges(ei, rows):
    e = ei.shape[1]
    pad = rows * IW - e
    src, dst = ei[0], ei[1]
    if pad:
        pad_src = (jnp.arange(pad, dtype=jnp.int32) * 41) % N
        src = jnp.concatenate([src, pad_src])
        dst = jnp.concatenate([dst, jnp.full((pad,), N, jnp.int32)])
    return src.reshape(rows, IW), dst.reshape(rows, IW)


def _encode_body(xt_ref, xc_ref, wt_ref, bt_ref, wc_ref, bc_ref, ht_ref, hc_ref):
    dot = functools.partial(jnp.dot, preferred_element_type=jnp.float32)
    ht = dot(xt_ref[...], wt_ref[...]) + bt_ref[...]
    hc = dot(xc_ref[...], wc_ref[...]) + bc_ref[...]
    ht_ref[0] = ht[:, :HH]
    ht_ref[1] = ht[:, HH:]
    hc_ref[0] = hc[:, :HH]
    hc_ref[1] = hc[:, HH:]


def _encode(x_table, x_column, enc_t_W, enc_t_b, enc_c_W, enc_c_b):
    xt = x_table.reshape(N, 1).astype(jnp.float32)
    xc = x_column.reshape(N, 1).astype(jnp.float32)
    bt = enc_t_b.reshape(1, H)
    bc = enc_c_b.reshape(1, H)
    out_shape = [jax.ShapeDtypeStruct((NC, N, HH), jnp.float32)] * 2
    return pl.pallas_call(
        _encode_body,
        grid=(NBLK,),
        in_specs=[
            pl.BlockSpec((BN, 1), lambda i: (i, 0)),
            pl.BlockSpec((BN, 1), lambda i: (i, 0)),
            pl.BlockSpec((1, H), lambda i: (0, 0)),
            pl.BlockSpec((1, H), lambda i: (0, 0)),
            pl.BlockSpec((1, H), lambda i: (0, 0)),
            pl.BlockSpec((1, H), lambda i: (0, 0)),
        ],
        out_specs=[
            pl.BlockSpec((NC, BN, HH), lambda i: (0, i, 0)),
            pl.BlockSpec((NC, BN, HH), lambda i: (0, i, 0)),
        ],
        out_shape=out_shape,
    )(xt, xc, enc_t_W, bt, enc_c_W, bc)


def _cat(ref):
    return jnp.concatenate([ref[0], ref[1]], axis=1)


def _combine_body(final, ht_ref, hc_ref, sc_ref, sf_ref, sr_ref, cnt_ref,
                  wl_ref, bl_ref, wr_ref, cwt_ref, cbt_ref, cwc_ref, cbc_ref,
                  ot_ref, oc_ref):
    ht = _cat(ht_ref)
    hc = _cat(hc_ref)
    cnt = cnt_ref[...]
    rc = 1.0 / jnp.maximum(cnt, 1.0)
    mc = _cat(sc_ref) * rc[:, 0:1]
    mf = _cat(sf_ref) * rc[:, 1:2]
    mr = _cat(sr_ref) * rc[:, 2:3]
    dot = functools.partial(jnp.dot, preferred_element_type=jnp.float32)
    col_new = 0.5 * (dot(mc, wl_ref[0]) + dot(mf, wl_ref[1])
                     + dot(hc, wr_ref[0] + wr_ref[1])
                     + bl_ref[0, 0][None, :] + bl_ref[0, 1][None, :])
    tab_new = (dot(mr, wl_ref[2]) + dot(ht, wr_ref[2]) + bl_ref[0, 2][None, :])
    ht_new = jnp.maximum(ht + col_new, 0.0)
    hc_new = jnp.maximum(hc + tab_new, 0.0)
    if final:
        to = dot(ht_new, cwt_ref[...]) + cbt_ref[...]
        co = dot(hc_new, cwc_ref[...]) + cbc_ref[...]
        ot_ref[...] = 1.0 / (1.0 + jnp.exp(-to))
        oc_ref[...] = 1.0 / (1.0 + jnp.exp(-co))
    else:
        ot_ref[0] = ht_new[:, :HH]
        ot_ref[1] = ht_new[:, HH:]
        oc_ref[0] = hc_new[:, :HH]
        oc_ref[1] = hc_new[:, HH:]


def _combine(final, ht, hc, sum_c, sum_f, sum_r, cnt, Wl_l, bl_l, Wr_l,
             cls_t_W, cls_t_b, cls_c_W, cls_c_b):
    half_spec = pl.BlockSpec((NC, BN, HH), lambda i: (0, i, 0))
    if final:
        out_shape = [jax.ShapeDtypeStruct((N, 1), jnp.float32)] * 2
        out_specs = [pl.BlockSpec((BN, 1), lambda i: (i, 0))] * 2
    else:
        out_shape = [jax.ShapeDtypeStruct((NC, N, HH), jnp.float32)] * 2
        out_specs = [half_spec] * 2
    bl2 = bl_l.reshape(1, 3, H)
    return pl.pallas_call(
        functools.partial(_combine_body, final),
        grid=(NBLK,),
        in_specs=[
            half_spec, half_spec, half_spec, half_spec, half_spec,
            pl.BlockSpec((BN, 3), lambda i: (i, 0)),
            pl.BlockSpec((3, H, H), lambda i: (0, 0, 0)),
            pl.BlockSpec((1, 3, H), lambda i: (0, 0, 0)),
            pl.BlockSpec((3, H, H), lambda i: (0, 0, 0)),
            pl.BlockSpec((H, 1), lambda i: (0, 0)),
            pl.BlockSpec((1, 1), lambda i: (0, 0)),
            pl.BlockSpec((H, 1), lambda i: (0, 0)),
            pl.BlockSpec((1, 1), lambda i: (0, 0)),
        ],
        out_specs=out_specs,
        out_shape=out_shape,
    )(ht, hc, sum_c, sum_f, sum_r, cnt, Wl_l, bl2, Wr_l,
      cls_t_W, cls_t_b.reshape(1, 1), cls_c_W, cls_c_b.reshape(1, 1))


def kernel(x_table, x_column, ei_contains, ei_foreign_key, ei_rev_contains,
           enc_t_W, enc_t_b, enc_c_W, enc_c_b, Wl, bl, Wr,
           cls_t_W, cls_t_b, cls_c_W, cls_c_b):
    ht2, hc2 = _encode(x_table, x_column, enc_t_W, enc_t_b, enc_c_W, enc_c_b)
    ec = _prep_edges(ei_contains, R_C)
    ef = _prep_edges(ei_foreign_key, R_FK)
    er = _prep_edges(ei_rev_contains, R_C)
    cnt = None
    for l in range(L):
        if l == 0:
            sum_c, sum_f, sum_r, cnt_flat = _sc_agg(True, ht2, hc2, ec, ef, er)
            cnt = cnt_flat.reshape(3, N).T
        else:
            sum_c, sum_f, sum_r = _sc_agg(False, ht2, hc2, ec, ef, er)
        ht2, hc2 = _combine(l == L - 1, ht2, hc2, sum_c, sum_f, sum_r, cnt,
                            Wl[l], bl[l], Wr[l],
                            cls_t_W, cls_t_b, cls_c_W, cls_c_b)
    return (ht2[:, 0], hc2[:, 0])
